# Initial kernel scaffold; baseline (speedup 1.0000x reference)
import jax
import jax.numpy as jnp
from jax.experimental import pallas as pl


def kernel(w):
    raise NotImplementedError("write your pallas kernel here")


# speedup vs baseline: 1.6790x; 1.6790x over previous
import functools

import jax
import jax.numpy as jnp
import numpy as np
from jax.experimental import pallas as pl
from jax.experimental.pallas import tpu as pltpu

_CODE_VALUES = np.array([-3.0, -1.0, 1.0, 3.0], dtype=np.float32)


def _codebook_np():
    rs = np.random.RandomState(0)
    idx = rs.randint(0, 4, size=(512, 8))
    return _CODE_VALUES[idx]


def _build_consts():
    cb = _codebook_np()
    cnorm = (cb * cb).sum(axis=1)
    w1 = np.zeros((32, 2048), dtype=np.float32)
    for g in range(4):
        w1[8 * g : 8 * g + 8, 512 * g : 512 * (g + 1)] = cb.T
    cn = np.tile(cnorm / 2.0, 4)[None, :].astype(np.float32)
    w2 = np.zeros((2048, 64), dtype=np.float32)
    for g in range(4):
        w2[512 * g : 512 * (g + 1), 8 * g : 8 * g + 8] = cb
        w2[512 * g : 512 * (g + 1), 32 + 8 * g : 32 + 8 * g + 8] = 1.0
    import ml_dtypes
    return w1, cn, w2.astype(ml_dtypes.bfloat16)


_W1, _CN, _W2 = _build_consts()

_TB = 1024
_NB = (4096 * 4096) // 32


def _quant_body(x_ref, w1_ref, cn_ref, w2_ref, o_ref):
    x = x_ref[...]
    d = jnp.maximum(jnp.max(jnp.abs(x), axis=-1, keepdims=True) / 3.0, 1e-8)
    sub = x / d
    score = jax.lax.dot_general(
        sub, w1_ref[...], (((1,), (0,)), ((), ())),
        preferred_element_type=jnp.float32,
        precision=jax.lax.Precision.HIGHEST,
    ) - cn_ref[...]
    s3 = score.reshape(x.shape[0], 4, 512)
    m = jnp.max(s3, axis=-1, keepdims=True)
    oh = (s3 == m).astype(jnp.bfloat16).reshape(x.shape[0], 2048)
    qc = jax.lax.dot_general(
        oh, w2_ref[...], (((1,), (0,)), ((), ())),
        preferred_element_type=jnp.float32,
    )
    deq = (qc[:, :32] / qc[:, 32:]) * d
    o_ref[...] = x + (deq - x)


@jax.jit
def kernel(w):
    wf = w.reshape(_NB, 32)
    grid = _NB // _TB
    out = pl.pallas_call(
        _quant_body,
        grid=(grid,),
        in_specs=[
            pl.BlockSpec((_TB, 32), lambda i: (i, 0)),
            pl.BlockSpec((32, 2048), lambda i: (0, 0)),
            pl.BlockSpec((1, 2048), lambda i: (0, 0)),
            pl.BlockSpec((2048, 64), lambda i: (0, 0)),
        ],
        out_specs=pl.BlockSpec((_TB, 32), lambda i: (i, 0)),
        out_shape=jax.ShapeDtypeStruct((_NB, 32), jnp.float32),
    )(wf, jnp.asarray(_W1), jnp.asarray(_CN), jnp.asarray(_W2))
    return out.reshape(w.shape)


# speedup vs baseline: 1.8649x; 1.1107x over previous
import functools

import jax
import jax.numpy as jnp
import numpy as np
from jax.experimental import pallas as pl
from jax.experimental.pallas import tpu as pltpu

_CODE_VALUES = np.array([-3.0, -1.0, 1.0, 3.0], dtype=np.float32)


def _codebook_np():
    rs = np.random.RandomState(0)
    idx = rs.randint(0, 4, size=(512, 8))
    return _CODE_VALUES[idx]


def _build_consts():
    cb = _codebook_np()
    cnorm = (cb * cb).sum(axis=1)
    w1 = np.zeros((32, 2048), dtype=np.float32)
    for g in range(4):
        w1[8 * g : 8 * g + 8, 512 * g : 512 * (g + 1)] = cb.T
    cn = np.tile(cnorm, 4)[None, :].astype(np.float32)
    w2 = np.zeros((2048, 32), dtype=np.float32)
    for g in range(4):
        w2[512 * g : 512 * (g + 1), 8 * g : 8 * g + 8] = cb
    import ml_dtypes
    return w1.astype(ml_dtypes.bfloat16), cn, w2.astype(ml_dtypes.bfloat16)


_W1, _CN, _W2 = _build_consts()

_TB = 1024
_NB = (4096 * 4096) // 32


def _quant_body(x_ref, w1_ref, cn_ref, w2_ref, o_ref):
    x = x_ref[...]
    d = jnp.maximum(jnp.max(jnp.abs(x), axis=-1, keepdims=True) / 3.0, 1e-8)
    sub = x / d
    dots = jax.lax.dot_general(
        sub.astype(jnp.bfloat16), w1_ref[...], (((1,), (0,)), ((), ())),
        preferred_element_type=jnp.float32,
    )
    dist = cn_ref[...] - 2.0 * dots
    s3 = dist.reshape(x.shape[0], 4, 512)
    m = jnp.min(s3, axis=-1, keepdims=True)
    iota = jax.lax.broadcasted_iota(jnp.int32, s3.shape, 2)
    cand = jnp.where(s3 == m, iota, jnp.int32(1 << 30))
    idx = jnp.min(cand, axis=-1, keepdims=True)
    oh = (iota == idx).astype(jnp.bfloat16).reshape(x.shape[0], 2048)
    q = jax.lax.dot_general(
        oh, w2_ref[...], (((1,), (0,)), ((), ())),
        preferred_element_type=jnp.float32,
    )
    deq = q * d
    o_ref[...] = x + (deq - x)


@jax.jit
def kernel(w):
    wf = w.reshape(_NB, 32)
    grid = _NB // _TB
    out = pl.pallas_call(
        _quant_body,
        grid=(grid,),
        in_specs=[
            pl.BlockSpec((_TB, 32), lambda i: (i, 0)),
            pl.BlockSpec((32, 2048), lambda i: (0, 0)),
            pl.BlockSpec((1, 2048), lambda i: (0, 0)),
            pl.BlockSpec((2048, 32), lambda i: (0, 0)),
        ],
        out_specs=pl.BlockSpec((_TB, 32), lambda i: (i, 0)),
        out_shape=jax.ShapeDtypeStruct((_NB, 32), jnp.float32),
    )(wf, jnp.asarray(_W1), jnp.asarray(_CN), jnp.asarray(_W2))
    return out.reshape(w.shape)


# speedup vs baseline: 5.9033x; 3.1656x over previous
import functools

import jax
import jax.numpy as jnp
import numpy as np
from jax.experimental import pallas as pl
from jax.experimental.pallas import tpu as pltpu

_CODE_VALUES = np.array([-3.0, -1.0, 1.0, 3.0], dtype=np.float32)


def _codebook_np():
    rs = np.random.RandomState(0)
    idx = rs.randint(0, 4, size=(512, 8))
    return _CODE_VALUES[idx]


def _build_consts():
    import ml_dtypes
    cb = _codebook_np()
    cnorm = (cb * cb).sum(axis=1)
    w1t = np.zeros((2048, 32), dtype=np.float32)
    w2t = np.zeros((32, 2048), dtype=np.float32)
    for g in range(4):
        w1t[512 * g : 512 * (g + 1), 8 * g : 8 * g + 8] = cb
        w2t[8 * g : 8 * g + 8, 512 * g : 512 * (g + 1)] = cb.T
    cn = np.tile(cnorm, 4)[:, None].astype(np.float32)
    return (w1t.astype(ml_dtypes.bfloat16), cn, w2t.astype(ml_dtypes.bfloat16))


_W1T, _CN, _W2T = _build_consts()

_TB = 2048
_NB = (4096 * 4096) // 32
_BIG = np.int32(1 << 30)


def _quant_body(xt_ref, w1t_ref, cn_ref, w2t_ref, o_ref):
    xt = xt_ref[...]
    d = jnp.maximum(jnp.max(jnp.abs(xt), axis=0, keepdims=True) / 3.0, 1e-8)
    sub = (xt / d).astype(jnp.bfloat16)
    dots = jax.lax.dot_general(
        w1t_ref[...], sub, (((1,), (0,)), ((), ())),
        preferred_element_type=jnp.float32,
    )
    dist = cn_ref[...] - 2.0 * dots
    iota = jax.lax.broadcasted_iota(jnp.int32, (512, dist.shape[1]), 0)
    ohs = []
    for g in range(4):
        sg = dist[512 * g : 512 * (g + 1), :]
        m = jnp.min(sg, axis=0, keepdims=True)
        cand = jnp.where(sg == m, iota, _BIG)
        idx = jnp.min(cand, axis=0, keepdims=True)
        ohs.append((iota == idx).astype(jnp.bfloat16))
    oh = jnp.concatenate(ohs, axis=0)
    q = jax.lax.dot_general(
        w2t_ref[...], oh, (((1,), (0,)), ((), ())),
        preferred_element_type=jnp.float32,
    )
    deq = q * d
    o_ref[...] = xt + (deq - xt)


@jax.jit
def kernel(w):
    xt = w.reshape(_NB, 32).T
    grid = _NB // _TB
    out = pl.pallas_call(
        _quant_body,
        grid=(grid,),
        in_specs=[
            pl.BlockSpec((32, _TB), lambda i: (0, i)),
            pl.BlockSpec((2048, 32), lambda i: (0, 0)),
            pl.BlockSpec((2048, 1), lambda i: (0, 0)),
            pl.BlockSpec((32, 2048), lambda i: (0, 0)),
        ],
        out_specs=pl.BlockSpec((32, _TB), lambda i: (0, i)),
        out_shape=jax.ShapeDtypeStruct((32, _NB), jnp.float32),
    )(xt, jnp.asarray(_W1T), jnp.asarray(_CN), jnp.asarray(_W2T))
    return out.T.reshape(w.shape)


# speedup vs baseline: 6.7038x; 1.1356x over previous
import functools

import jax
import jax.numpy as jnp
import numpy as np
from jax.experimental import pallas as pl
from jax.experimental.pallas import tpu as pltpu

_CODE_VALUES = np.array([-3.0, -1.0, 1.0, 3.0], dtype=np.float32)


def _codebook_np():
    rs = np.random.RandomState(0)
    idx = rs.randint(0, 4, size=(512, 8))
    return _CODE_VALUES[idx]


def _build_consts():
    import ml_dtypes
    cb = _codebook_np()
    cnorm = (cb * cb).sum(axis=1)
    w1t = np.zeros((2048, 32), dtype=np.float32)
    w2t = np.zeros((32, 2048), dtype=np.float32)
    for g in range(4):
        w1t[512 * g : 512 * (g + 1), 8 * g : 8 * g + 8] = 2.0 * cb
        w2t[8 * g : 8 * g + 8, 512 * g : 512 * (g + 1)] = cb.T
    cn = np.tile(cnorm, 4)[:, None].astype(np.float32)
    return (w1t.astype(ml_dtypes.bfloat16), cn, w2t.astype(ml_dtypes.bfloat16))


_W1T, _CN, _W2T = _build_consts()

_TB = 2048
_NB = (4096 * 4096) // 32
_BIG = np.float32(1e9)


def _quant_body(x_ref, w1t_ref, cn_ref, w2t_ref, o_ref):
    xt = x_ref[...].T
    d = jnp.maximum(jnp.max(jnp.abs(xt), axis=0, keepdims=True) / 3.0, 1e-8)
    sub = (xt / d).astype(jnp.bfloat16)
    dots2 = jax.lax.dot_general(
        w1t_ref[...], sub, (((1,), (0,)), ((), ())),
        preferred_element_type=jnp.float32,
    )
    dist = cn_ref[...] - dots2
    iota = jax.lax.broadcasted_iota(
        jnp.int32, (512, dist.shape[1]), 0).astype(jnp.float32)
    ohs = []
    for g in range(4):
        sg = dist[512 * g : 512 * (g + 1), :]
        m = jnp.min(sg, axis=0, keepdims=True)
        cand = jnp.where(sg == m, iota, _BIG)
        idx = jnp.min(cand, axis=0, keepdims=True)
        ohs.append((iota == idx).astype(jnp.bfloat16))
    oh = jnp.concatenate(ohs, axis=0)
    q = jax.lax.dot_general(
        w2t_ref[...], oh, (((1,), (0,)), ((), ())),
        preferred_element_type=jnp.float32,
    )
    deq = q * d
    o_ref[...] = (xt + (deq - xt)).T


@jax.jit
def kernel(w):
    wf = w.reshape(_NB, 32)
    grid = _NB // _TB
    out = pl.pallas_call(
        _quant_body,
        grid=(grid,),
        in_specs=[
            pl.BlockSpec((_TB, 32), lambda i: (i, 0)),
            pl.BlockSpec((2048, 32), lambda i: (0, 0)),
            pl.BlockSpec((2048, 1), lambda i: (0, 0)),
            pl.BlockSpec((32, 2048), lambda i: (0, 0)),
        ],
        out_specs=pl.BlockSpec((_TB, 32), lambda i: (i, 0)),
        out_shape=jax.ShapeDtypeStruct((_NB, 32), jnp.float32),
    )(wf, jnp.asarray(_W1T), jnp.asarray(_CN), jnp.asarray(_W2T))
    return out.reshape(w.shape)


# speedup vs baseline: 7.9082x; 1.1797x over previous
import functools

import jax
import jax.numpy as jnp
import numpy as np
from jax.experimental import pallas as pl
from jax.experimental.pallas import tpu as pltpu

_CODE_VALUES = np.array([-3.0, -1.0, 1.0, 3.0], dtype=np.float32)


def _codebook_np():
    rs = np.random.RandomState(0)
    idx = rs.randint(0, 4, size=(512, 8))
    return _CODE_VALUES[idx]


def _build_consts():
    import ml_dtypes
    cb = _codebook_np()
    cnorm = (cb * cb).sum(axis=1)
    w1t = np.zeros((2048, 32), dtype=np.float32)
    w2t = np.zeros((32, 2048), dtype=np.float32)
    for g in range(4):
        w1t[512 * g : 512 * (g + 1), 8 * g : 8 * g + 8] = 2.0 * cb
        w2t[8 * g : 8 * g + 8, 512 * g : 512 * (g + 1)] = cb.T
    cn = np.tile(cnorm, 4)[:, None].astype(np.float32)
    return (w1t.astype(ml_dtypes.bfloat16), cn, w2t.astype(ml_dtypes.bfloat16))


_W1T, _CN, _W2T = _build_consts()

_TB = 2048
_NB = (4096 * 4096) // 32
_BIG = np.float32(1e9)


def _quant_body(x_ref, w1t_ref, cn_ref, w2t_ref, o_ref):
    x = x_ref[...]
    xT3 = x.T.reshape(4, 32, x.shape[0])
    xt = jnp.concatenate([xT3[0], xT3[1], xT3[2], xT3[3]], axis=1)
    d = jnp.maximum(jnp.max(jnp.abs(xt), axis=0, keepdims=True) / 3.0, 1e-8)
    sub = (xt / d).astype(jnp.bfloat16)
    dots2 = jax.lax.dot_general(
        w1t_ref[...], sub, (((1,), (0,)), ((), ())),
        preferred_element_type=jnp.float32,
    )
    dist = cn_ref[...] - dots2
    iota = jax.lax.broadcasted_iota(
        jnp.int32, (512, dist.shape[1]), 0).astype(jnp.float32)
    ohs = []
    for g in range(4):
        sg = dist[512 * g : 512 * (g + 1), :]
        m = jnp.min(sg, axis=0, keepdims=True)
        cand = jnp.where(sg == m, iota, _BIG)
        idx = jnp.min(cand, axis=0, keepdims=True)
        ohs.append((iota == idx).astype(jnp.bfloat16))
    oh = jnp.concatenate(ohs, axis=0)
    q = jax.lax.dot_general(
        w2t_ref[...], oh, (((1,), (0,)), ((), ())),
        preferred_element_type=jnp.float32,
    )
    deq = q * d
    out = xt + (deq - xt)
    tr = x.shape[0]
    o_ref[...] = jnp.concatenate(
        [out[:, g * tr : (g + 1) * tr] for g in range(4)], axis=0).T


@jax.jit
def kernel(w):
    nr = _NB // 4
    tr = _TB // 4
    wf = w.reshape(nr, 128)
    grid = nr // tr
    out = pl.pallas_call(
        _quant_body,
        grid=(grid,),
        in_specs=[
            pl.BlockSpec((tr, 128), lambda i: (i, 0)),
            pl.BlockSpec((2048, 32), lambda i: (0, 0)),
            pl.BlockSpec((2048, 1), lambda i: (0, 0)),
            pl.BlockSpec((32, 2048), lambda i: (0, 0)),
        ],
        out_specs=pl.BlockSpec((tr, 128), lambda i: (i, 0)),
        out_shape=jax.ShapeDtypeStruct((nr, 128), jnp.float32),
    )(wf, jnp.asarray(_W1T), jnp.asarray(_CN), jnp.asarray(_W2T))
    return out.reshape(w.shape)


# speedup vs baseline: 8.2821x; 1.0473x over previous
import functools

import jax
import jax.numpy as jnp
import numpy as np
from jax.experimental import pallas as pl
from jax.experimental.pallas import tpu as pltpu

_CODE_VALUES = np.array([-3.0, -1.0, 1.0, 3.0], dtype=np.float32)


def _codebook_np():
    rs = np.random.RandomState(0)
    idx = rs.randint(0, 4, size=(512, 8))
    return _CODE_VALUES[idx]


def _build_consts():
    import ml_dtypes
    cb = _codebook_np()
    cnorm = (cb * cb).sum(axis=1)
    w1t = np.zeros((2048, 33), dtype=np.float32)
    w2t = np.zeros((32, 2048), dtype=np.float32)
    for g in range(4):
        w1t[512 * g : 512 * (g + 1), 8 * g : 8 * g + 8] = -2.0 * cb
        w1t[512 * g : 512 * (g + 1), 32] = cnorm
        w2t[8 * g : 8 * g + 8, 512 * g : 512 * (g + 1)] = cb.T
    return (w1t.astype(ml_dtypes.bfloat16), w2t.astype(ml_dtypes.bfloat16))


_W1T, _W2T = _build_consts()

_TB = 2048
_NB = (4096 * 4096) // 32
_BIG = np.float32(1e9)


def _quant_body(x_ref, w1t_ref, w2t_ref, o_ref):
    x = x_ref[...]
    xT3 = x.T.reshape(4, 32, x.shape[0])
    xt = jnp.concatenate([xT3[0], xT3[1], xT3[2], xT3[3]], axis=1)
    d = jnp.maximum(jnp.max(jnp.abs(xt), axis=0, keepdims=True) / 3.0, 1e-8)
    sub = (xt / d).astype(jnp.bfloat16)
    sub1 = jnp.concatenate(
        [sub, jnp.ones((1, sub.shape[1]), jnp.bfloat16)], axis=0)
    dist = jax.lax.dot_general(
        w1t_ref[...], sub1, (((1,), (0,)), ((), ())),
        preferred_element_type=jnp.float32,
    )
    iota = jax.lax.broadcasted_iota(
        jnp.int32, (512, dist.shape[1]), 0).astype(jnp.float32)
    ohs = []
    for g in range(4):
        sg = dist[512 * g : 512 * (g + 1), :]
        m = jnp.min(sg, axis=0, keepdims=True)
        cand = jnp.where(sg == m, iota, _BIG)
        idx = jnp.min(cand, axis=0, keepdims=True)
        ohs.append((iota == idx).astype(jnp.bfloat16))
    oh = jnp.concatenate(ohs, axis=0)
    q = jax.lax.dot_general(
        w2t_ref[...], oh, (((1,), (0,)), ((), ())),
        preferred_element_type=jnp.float32,
    )
    deq = q * d
    out = xt + (deq - xt)
    tr = x.shape[0]
    o_ref[...] = jnp.concatenate(
        [out[:, g * tr : (g + 1) * tr] for g in range(4)], axis=0).T


@jax.jit
def kernel(w):
    nr = _NB // 4
    tr = _TB // 4
    wf = w.reshape(nr, 128)
    grid = nr // tr
    out = pl.pallas_call(
        _quant_body,
        grid=(grid,),
        in_specs=[
            pl.BlockSpec((tr, 128), lambda i: (i, 0)),
            pl.BlockSpec((2048, 33), lambda i: (0, 0)),
            pl.BlockSpec((32, 2048), lambda i: (0, 0)),
        ],
        out_specs=pl.BlockSpec((tr, 128), lambda i: (i, 0)),
        out_shape=jax.ShapeDtypeStruct((nr, 128), jnp.float32),
    )(wf, jnp.asarray(_W1T), jnp.asarray(_W2T))
    return out.reshape(w.shape)


# speedup vs baseline: 8.3606x; 1.0095x over previous
import functools

import jax
import jax.numpy as jnp
import numpy as np
from jax.experimental import pallas as pl
from jax.experimental.pallas import tpu as pltpu

_CODE_VALUES = np.array([-3.0, -1.0, 1.0, 3.0], dtype=np.float32)


def _codebook_np():
    rs = np.random.RandomState(0)
    idx = rs.randint(0, 4, size=(512, 8))
    return _CODE_VALUES[idx]


def _build_consts():
    import ml_dtypes
    cb = _codebook_np()
    cnorm = (cb * cb).sum(axis=1)
    w1t = np.zeros((2048, 33), dtype=np.float32)
    w2t = np.zeros((32, 2048), dtype=np.float32)
    for g in range(4):
        w1t[512 * g : 512 * (g + 1), 8 * g : 8 * g + 8] = -2.0 * cb
        w1t[512 * g : 512 * (g + 1), 32] = cnorm
        w2t[8 * g : 8 * g + 8, 512 * g : 512 * (g + 1)] = cb.T
    return (w1t.astype(ml_dtypes.bfloat16), w2t.astype(ml_dtypes.bfloat16))


_W1T, _W2T = _build_consts()

_TB = 2048
_NB = (4096 * 4096) // 32
_BIG = np.float32(1e9)


def _quant_body(x_ref, w1t_ref, w2t_ref, o_ref):
    x = x_ref[...]
    xT3 = x.T.reshape(4, 32, x.shape[0])
    xt = jnp.concatenate([xT3[0], xT3[1], xT3[2], xT3[3]], axis=1)
    d = jnp.maximum(jnp.max(jnp.abs(xt), axis=0, keepdims=True) / 3.0, 1e-8)
    sub = (xt / d).astype(jnp.bfloat16)
    sub1 = jnp.concatenate(
        [sub, jnp.ones((1, sub.shape[1]), jnp.bfloat16)], axis=0)
    dist = jax.lax.dot_general(
        w1t_ref[...], sub1, (((1,), (0,)), ((), ())),
        preferred_element_type=jnp.float32,
    )
    tb = dist.shape[1]
    iota = jax.lax.broadcasted_iota(
        jnp.int32, (512, tb), 0).astype(jnp.float32)
    ohs = []
    for g in range(4):
        sg = dist[512 * g : 512 * (g + 1), :]
        m = jnp.min(sg, axis=0, keepdims=True)
        acc = jnp.full((8, tb), _BIG, jnp.float32)
        for k in range(64):
            ck = jnp.where(sg[8 * k : 8 * k + 8, :] == m,
                           iota[8 * k : 8 * k + 8, :], _BIG)
            acc = jnp.minimum(acc, ck)
        idx = jnp.min(acc, axis=0, keepdims=True)
        ohs.append((iota == idx).astype(jnp.bfloat16))
    oh = jnp.concatenate(ohs, axis=0)
    q = jax.lax.dot_general(
        w2t_ref[...], oh, (((1,), (0,)), ((), ())),
        preferred_element_type=jnp.float32,
    )
    deq = q * d
    out = xt + (deq - xt)
    tr = x.shape[0]
    o_ref[...] = jnp.concatenate(
        [out[:, g * tr : (g + 1) * tr] for g in range(4)], axis=0).T


@jax.jit
def kernel(w):
    nr = _NB // 4
    tr = _TB // 4
    wf = w.reshape(nr, 128)
    grid = nr // tr
    out = pl.pallas_call(
        _quant_body,
        grid=(grid,),
        in_specs=[
            pl.BlockSpec((tr, 128), lambda i: (i, 0)),
            pl.BlockSpec((2048, 33), lambda i: (0, 0)),
            pl.BlockSpec((32, 2048), lambda i: (0, 0)),
        ],
        out_specs=pl.BlockSpec((tr, 128), lambda i: (i, 0)),
        out_shape=jax.ShapeDtypeStruct((nr, 128), jnp.float32),
    )(wf, jnp.asarray(_W1T), jnp.asarray(_W2T))
    return out.reshape(w.shape)


# speedup vs baseline: 8.6877x; 1.0391x over previous
import functools

import jax
import jax.numpy as jnp
import numpy as np
from jax.experimental import pallas as pl
from jax.experimental.pallas import tpu as pltpu

_CODE_VALUES = np.array([-3.0, -1.0, 1.0, 3.0], dtype=np.float32)


def _codebook_np():
    rs = np.random.RandomState(0)
    idx = rs.randint(0, 4, size=(512, 8))
    return _CODE_VALUES[idx]


def _build_consts():
    import ml_dtypes
    cb = _codebook_np()
    cnorm = (cb * cb).sum(axis=1)
    w1t = np.zeros((2048, 33), dtype=np.float32)
    w2t = np.zeros((32, 2048), dtype=np.float32)
    for g in range(4):
        w1t[512 * g : 512 * (g + 1), 8 * g : 8 * g + 8] = -2.0 * cb
        w1t[512 * g : 512 * (g + 1), 32] = cnorm
        w2t[8 * g : 8 * g + 8, 512 * g : 512 * (g + 1)] = cb.T
    return (w1t.astype(ml_dtypes.bfloat16), w2t.astype(ml_dtypes.bfloat16))


_W1T, _W2T = _build_consts()

_TB = 4096
_NB = (4096 * 4096) // 32
_BIG = np.float32(1e9)


def _quant_body(x_ref, w1t_ref, w2t_ref, o_ref):
    x = x_ref[...]
    xT3 = x.T.reshape(4, 32, x.shape[0])
    xt = jnp.concatenate([xT3[0], xT3[1], xT3[2], xT3[3]], axis=1)
    d = jnp.maximum(jnp.max(jnp.abs(xt), axis=0, keepdims=True) / 3.0, 1e-8)
    sub = (xt / d).astype(jnp.bfloat16)
    sub1 = jnp.concatenate(
        [sub, jnp.ones((1, sub.shape[1]), jnp.bfloat16)], axis=0)
    dist = jax.lax.dot_general(
        w1t_ref[...], sub1, (((1,), (0,)), ((), ())),
        preferred_element_type=jnp.float32,
    )
    tb = dist.shape[1]
    iota = jax.lax.broadcasted_iota(
        jnp.int32, (512, tb), 0).astype(jnp.float32)
    ohs = []
    for g in range(4):
        sg = dist[512 * g : 512 * (g + 1), :]
        m = jnp.min(sg, axis=0, keepdims=True)
        acc = jnp.full((8, tb), _BIG, jnp.float32)
        for k in range(64):
            ck = jnp.where(sg[8 * k : 8 * k + 8, :] == m,
                           iota[8 * k : 8 * k + 8, :], _BIG)
            acc = jnp.minimum(acc, ck)
        idx = jnp.min(acc, axis=0, keepdims=True)
        ohs.append((iota == idx).astype(jnp.bfloat16))
    oh = jnp.concatenate(ohs, axis=0)
    q = jax.lax.dot_general(
        w2t_ref[...], oh, (((1,), (0,)), ((), ())),
        preferred_element_type=jnp.float32,
    )
    deq = q * d
    out = xt + (deq - xt)
    tr = x.shape[0]
    o_ref[...] = jnp.concatenate(
        [out[:, g * tr : (g + 1) * tr] for g in range(4)], axis=0).T


@jax.jit
def kernel(w):
    nr = _NB // 4
    tr = _TB // 4
    wf = w.reshape(nr, 128)
    grid = nr // tr
    out = pl.pallas_call(
        _quant_body,
        grid=(grid,),
        in_specs=[
            pl.BlockSpec((tr, 128), lambda i: (i, 0)),
            pl.BlockSpec((2048, 33), lambda i: (0, 0)),
            pl.BlockSpec((32, 2048), lambda i: (0, 0)),
        ],
        out_specs=pl.BlockSpec((tr, 128), lambda i: (i, 0)),
        out_shape=jax.ShapeDtypeStruct((nr, 128), jnp.float32),
    )(wf, jnp.asarray(_W1T), jnp.asarray(_W2T))
    return out.reshape(w.shape)
